# Initial kernel scaffold; baseline (speedup 1.0000x reference)
import jax
import jax.numpy as jnp
from jax.experimental import pallas as pl


def kernel(x, edge_index, W1, b1, W2, b2):
    raise NotImplementedError("write your pallas kernel here")


# speedup vs baseline: 24.3326x; 24.3326x over previous
import functools

import jax
import jax.numpy as jnp
from jax import lax
from jax.experimental import pallas as pl
from jax.experimental.pallas import tpu as pltpu
from jax.experimental.pallas import tpu_sc as plsc

N = 10000
E = 320000
D_IN = 128
D_HID = 64
D_OUT = 128

NC = 2
NS = 16
NW = NC * NS
EPW = E // NW
K = 80
C = EPW // K
NP = 10240
RPT = NP // NS

_mesh = plsc.VectorSubcoreMesh(core_axis_name="c", subcore_axis_name="s")
_sc_params = pltpu.CompilerParams(use_tc_tiling_on_sc=False)


DW = 16


@functools.partial(
    pl.kernel,
    out_type=jax.ShapeDtypeStruct((NC, NP, DW), jnp.float32),
    mesh=_mesh,
    scratch_types=[
        pltpu.VMEM((C, K), jnp.int32),
        pltpu.VMEM((K, DW), jnp.float32),
        pltpu.VMEM_SHARED((NP, DW), jnp.float32),
        pltpu.SemaphoreType.DMA,
    ],
    compiler_params=_sc_params,
)
def _deg_sc(dst_hbm, ones_hbm, zeros_hbm, out_hbm, dst_v, ones_v, acc, sem):
    c = lax.axis_index("c")
    s = lax.axis_index("s")
    wid = c * NS + s
    base = s * RPT
    pltpu.sync_copy(zeros_hbm.at[pl.ds(base, RPT)], acc.at[pl.ds(base, RPT)])
    pltpu.sync_copy(ones_hbm, ones_v)
    pltpu.sync_copy(dst_hbm.at[wid], dst_v)
    plsc.subcore_barrier()

    def body(j, carry):
        pltpu.sync_copy(ones_v, acc.at[dst_v.at[j]], add=True)
        return carry

    lax.fori_loop(0, C, body, 0)
    plsc.subcore_barrier()
    pltpu.sync_copy(acc.at[pl.ds(base, RPT)], out_hbm.at[c, pl.ds(base, RPT)])


@functools.partial(
    pl.kernel,
    out_type=jax.ShapeDtypeStruct((NC, NP, D_HID), jnp.float32),
    mesh=_mesh,
    scratch_types=[
        pltpu.VMEM((C, K), jnp.int32),
        pltpu.VMEM((C, K), jnp.int32),
        pltpu.VMEM((K, D_HID), jnp.float32),
        pltpu.VMEM_SHARED((NP, D_HID), jnp.float32),
        pltpu.SemaphoreType.DMA,
    ],
    compiler_params=_sc_params,
)
def _scatter_sc(src_hbm, dst_hbm, y_hbm, zeros_hbm, out_hbm,
                src_v, dst_v, rows_v, acc, sem):
    c = lax.axis_index("c")
    s = lax.axis_index("s")
    wid = c * NS + s
    base = s * RPT
    pltpu.sync_copy(zeros_hbm.at[pl.ds(base, RPT)], acc.at[pl.ds(base, RPT)])
    pltpu.sync_copy(src_hbm.at[wid], src_v)
    pltpu.sync_copy(dst_hbm.at[wid], dst_v)
    plsc.subcore_barrier()

    def body(j, carry):
        pltpu.async_copy(y_hbm.at[src_v.at[j]], rows_v, sem).wait()
        pltpu.sync_copy(rows_v, acc.at[dst_v.at[j]], add=True)
        return carry

    lax.fori_loop(0, C, body, 0)
    plsc.subcore_barrier()
    pltpu.sync_copy(acc.at[pl.ds(base, RPT)], out_hbm.at[c, pl.ds(base, RPT)])


def _tc1_body(parts_ref, x_ref, w1_ref, dinv_ref, y1_ref):
    deg = parts_ref[0, :N, 0:1] + parts_ref[1, :N, 0:1] + 1.0
    dinv = lax.rsqrt(deg)
    dinv_ref[...] = dinv
    xw = jnp.dot(x_ref[...], w1_ref[...], preferred_element_type=jnp.float32)
    y1_ref[...] = dinv * xw


def _tc2_body(zp_ref, y1_ref, dinv_ref, b1_ref, y2_ref):
    dinv = dinv_ref[...]
    agg = zp_ref[0, :N, :] + zp_ref[1, :N, :] + y1_ref[...]
    h = jnp.maximum(dinv * agg + b1_ref[...], 0.0)
    y2_ref[...] = dinv * h


def _tc3_body(zp_ref, y2_ref, dinv_ref, w2_ref, b2_ref, out_ref):
    ah = dinv_ref[...] * (zp_ref[0, :N, :] + zp_ref[1, :N, :] + y2_ref[...])
    out_ref[...] = (
        jnp.dot(ah, w2_ref[...], preferred_element_type=jnp.float32)
        + b2_ref[...]
    )


_tc1 = pl.pallas_call(
    _tc1_body,
    out_shape=(
        jax.ShapeDtypeStruct((N, 1), jnp.float32),
        jax.ShapeDtypeStruct((N, D_HID), jnp.float32),
    ),
)
_tc2 = pl.pallas_call(
    _tc2_body,
    out_shape=jax.ShapeDtypeStruct((N, D_HID), jnp.float32),
)
_tc3 = pl.pallas_call(
    _tc3_body,
    out_shape=jax.ShapeDtypeStruct((N, D_OUT), jnp.float32),
)


def kernel(x, edge_index, W1, b1, W2, b2):
    src = edge_index[0].reshape(NW, C, K)
    dst = edge_index[1].reshape(NW, C, K)
    ones_col = jnp.ones((K, DW), jnp.float32)
    zeros_col = jnp.zeros((NP, DW), jnp.float32)
    zeros_rows = jnp.zeros((NP, D_HID), jnp.float32)

    deg_parts = _deg_sc(dst, ones_col, zeros_col)
    dinv, y1 = _tc1(deg_parts, x, W1)
    z1_parts = _scatter_sc(src, dst, y1, zeros_rows)
    y2 = _tc2(z1_parts, y1, dinv, b1.reshape(1, D_HID))
    z2_parts = _scatter_sc(src, dst, y2, zeros_rows)
    return _tc3(z2_parts, y2, dinv, W2, b2.reshape(1, D_OUT))


# speedup vs baseline: 35.2243x; 1.4476x over previous
import functools

import jax
import jax.numpy as jnp
from jax import lax
from jax.experimental import pallas as pl
from jax.experimental.pallas import tpu as pltpu
from jax.experimental.pallas import tpu_sc as plsc

N = 10000
E = 320000
D_IN = 128
D_HID = 64
D_OUT = 128

NC = 2
NS = 16
NW = NC * NS
EPW = E // NW
K = 80
C = EPW // K
NP = 10240
RPT = NP // NS

_mesh = plsc.VectorSubcoreMesh(core_axis_name="c", subcore_axis_name="s")
_sc_params = pltpu.CompilerParams(use_tc_tiling_on_sc=False)


DW = 16


@functools.partial(
    pl.kernel,
    out_type=jax.ShapeDtypeStruct((NC, NP, DW), jnp.float32),
    mesh=_mesh,
    scratch_types=[
        pltpu.VMEM((C, K), jnp.int32),
        pltpu.VMEM((K, DW), jnp.float32),
        pltpu.VMEM_SHARED((NP, DW), jnp.float32),
        pltpu.SemaphoreType.DMA,
    ],
    compiler_params=_sc_params,
)
def _deg_sc(dst_hbm, ones_hbm, zeros_hbm, out_hbm, dst_v, ones_v, acc, sem):
    c = lax.axis_index("c")
    s = lax.axis_index("s")
    wid = c * NS + s
    base = s * RPT
    pltpu.sync_copy(zeros_hbm.at[pl.ds(base, RPT)], acc.at[pl.ds(base, RPT)])
    pltpu.sync_copy(ones_hbm, ones_v)
    pltpu.sync_copy(dst_hbm.at[wid], dst_v)
    plsc.subcore_barrier()

    def body(j, carry):
        pltpu.sync_copy(ones_v, acc.at[dst_v.at[j]], add=True)
        return carry

    lax.fori_loop(0, C, body, 0)
    plsc.subcore_barrier()
    pltpu.sync_copy(acc.at[pl.ds(base, RPT)], out_hbm.at[c, pl.ds(base, RPT)])


@functools.partial(
    pl.kernel,
    out_type=jax.ShapeDtypeStruct((NC, NP, D_HID), jnp.float32),
    mesh=_mesh,
    scratch_types=[
        pltpu.VMEM((C, K), jnp.int32),
        pltpu.VMEM((C, K), jnp.int32),
        pltpu.VMEM((K, D_HID), jnp.float32),
        pltpu.VMEM((K, D_HID), jnp.float32),
        pltpu.VMEM_SHARED((NP, D_HID), jnp.float32),
        pltpu.SemaphoreType.DMA,
        pltpu.SemaphoreType.DMA,
    ],
    compiler_params=_sc_params,
)
def _scatter_sc(src_hbm, dst_hbm, y_hbm, zeros_hbm, out_hbm,
                src_v, dst_v, buf0, buf1, acc, sem0, sem1):
    c = lax.axis_index("c")
    s = lax.axis_index("s")
    wid = c * NS + s
    base = s * RPT
    pltpu.sync_copy(zeros_hbm.at[pl.ds(base, RPT)], acc.at[pl.ds(base, RPT)])
    pltpu.sync_copy(src_hbm.at[wid], src_v)
    pltpu.sync_copy(dst_hbm.at[wid], dst_v)
    plsc.subcore_barrier()

    pltpu.async_copy(y_hbm.at[src_v.at[0]], buf0, sem0)

    def body(t, carry):
        j0 = 2 * t
        j1 = 2 * t + 1
        jn = jnp.minimum(2 * t + 2, C - 1)
        pltpu.async_copy(y_hbm.at[src_v.at[j1]], buf1, sem1)
        pltpu.make_async_copy(y_hbm.at[src_v.at[j0]], buf0, sem0).wait()
        pltpu.sync_copy(buf0, acc.at[dst_v.at[j0]], add=True)
        pltpu.async_copy(y_hbm.at[src_v.at[jn]], buf0, sem0)
        pltpu.make_async_copy(y_hbm.at[src_v.at[j1]], buf1, sem1).wait()
        pltpu.sync_copy(buf1, acc.at[dst_v.at[j1]], add=True)
        return carry

    lax.fori_loop(0, C // 2, body, 0)
    pltpu.make_async_copy(y_hbm.at[src_v.at[C - 1]], buf0, sem0).wait()
    pltpu.sync_copy(buf0, acc.at[dst_v.at[C - 1]], add=True)
    plsc.subcore_barrier()
    pltpu.sync_copy(acc.at[pl.ds(base, RPT)], out_hbm.at[c, pl.ds(base, RPT)])


def _tc1_body(parts_ref, x_ref, w1_ref, dinv_ref, y1_ref):
    deg = parts_ref[0, :N, 0:1] + parts_ref[1, :N, 0:1] + 1.0
    dinv = lax.rsqrt(deg)
    dinv_ref[...] = dinv
    xw = jnp.dot(x_ref[...], w1_ref[...], preferred_element_type=jnp.float32)
    y1_ref[...] = dinv * xw


def _tc2_body(zp_ref, y1_ref, dinv_ref, b1_ref, y2_ref):
    dinv = dinv_ref[...]
    agg = zp_ref[0, :N, :] + zp_ref[1, :N, :] + y1_ref[...]
    h = jnp.maximum(dinv * agg + b1_ref[...], 0.0)
    y2_ref[...] = dinv * h


def _tc3_body(zp_ref, y2_ref, dinv_ref, w2_ref, b2_ref, out_ref):
    ah = dinv_ref[...] * (zp_ref[0, :N, :] + zp_ref[1, :N, :] + y2_ref[...])
    out_ref[...] = (
        jnp.dot(ah, w2_ref[...], preferred_element_type=jnp.float32)
        + b2_ref[...]
    )


_tc1 = pl.pallas_call(
    _tc1_body,
    out_shape=(
        jax.ShapeDtypeStruct((N, 1), jnp.float32),
        jax.ShapeDtypeStruct((N, D_HID), jnp.float32),
    ),
)
_tc2 = pl.pallas_call(
    _tc2_body,
    out_shape=jax.ShapeDtypeStruct((N, D_HID), jnp.float32),
)
_tc3 = pl.pallas_call(
    _tc3_body,
    out_shape=jax.ShapeDtypeStruct((N, D_OUT), jnp.float32),
)


def kernel(x, edge_index, W1, b1, W2, b2):
    src = edge_index[0].reshape(NW, C, K)
    dst = edge_index[1].reshape(NW, C, K)
    ones_col = jnp.ones((K, DW), jnp.float32)
    zeros_col = jnp.zeros((NP, DW), jnp.float32)
    zeros_rows = jnp.zeros((NP, D_HID), jnp.float32)

    deg_parts = _deg_sc(dst, ones_col, zeros_col)
    dinv, y1 = _tc1(deg_parts, x, W1)
    z1_parts = _scatter_sc(src, dst, y1, zeros_rows)
    y2 = _tc2(z1_parts, y1, dinv, b1.reshape(1, D_HID))
    z2_parts = _scatter_sc(src, dst, y2, zeros_rows)
    return _tc3(z2_parts, y2, dinv, W2, b2.reshape(1, D_OUT))


# speedup vs baseline: 41.2316x; 1.1705x over previous
import functools

import jax
import jax.numpy as jnp
from jax import lax
from jax.experimental import pallas as pl
from jax.experimental.pallas import tpu as pltpu
from jax.experimental.pallas import tpu_sc as plsc

N = 10000
E = 320000
D_IN = 128
D_HID = 64
D_OUT = 128

NC = 2
NS = 16
NW = NC * NS
EPW = E // NW
K = 80
C = EPW // K
NP = 10240
RPT = NP // NS

_mesh = plsc.VectorSubcoreMesh(core_axis_name="c", subcore_axis_name="s")
_sc_params = pltpu.CompilerParams(use_tc_tiling_on_sc=False)


DW = 16


@functools.partial(
    pl.kernel,
    out_type=jax.ShapeDtypeStruct((NC, NP, DW), jnp.float32),
    mesh=_mesh,
    scratch_types=[
        pltpu.VMEM((C, K), jnp.int32),
        pltpu.VMEM((K, DW), jnp.float32),
        pltpu.VMEM_SHARED((NP, DW), jnp.float32),
        pltpu.SemaphoreType.DMA,
    ],
    compiler_params=_sc_params,
)
def _deg_sc(dst_hbm, ones_hbm, zeros_hbm, out_hbm, dst_v, ones_v, acc, sem):
    c = lax.axis_index("c")
    s = lax.axis_index("s")
    wid = c * NS + s
    base = s * RPT
    pltpu.sync_copy(zeros_hbm.at[pl.ds(base, RPT)], acc.at[pl.ds(base, RPT)])
    pltpu.sync_copy(ones_hbm, ones_v)
    pltpu.sync_copy(dst_hbm.at[wid], dst_v)
    plsc.subcore_barrier()

    def body(j, carry):
        pltpu.sync_copy(ones_v, acc.at[dst_v.at[j]], add=True)
        return carry

    lax.fori_loop(0, C, body, 0)
    plsc.subcore_barrier()
    pltpu.sync_copy(acc.at[pl.ds(base, RPT)], out_hbm.at[c, pl.ds(base, RPT)])


@functools.partial(
    pl.kernel,
    out_type=jax.ShapeDtypeStruct((NC, NP, D_HID), jnp.float32),
    mesh=_mesh,
    scratch_types=[
        pltpu.VMEM((C, K), jnp.int32),
        pltpu.VMEM((C, K), jnp.int32),
        [pltpu.VMEM((K, D_HID), jnp.float32) for _ in range(5)],
        pltpu.VMEM_SHARED((NP, D_HID), jnp.float32),
        [pltpu.SemaphoreType.DMA for _ in range(5)],
        [pltpu.SemaphoreType.DMA for _ in range(5)],
    ],
    compiler_params=_sc_params,
)
def _scatter_sc(src_hbm, dst_hbm, y_hbm, zeros_hbm, out_hbm,
                src_v, dst_v, bufs, acc, gsems, ssems):
    c = lax.axis_index("c")
    s = lax.axis_index("s")
    wid = c * NS + s
    base = s * RPT
    pltpu.sync_copy(zeros_hbm.at[pl.ds(base, RPT)], acc.at[pl.ds(base, RPT)])
    pltpu.sync_copy(src_hbm.at[wid], src_v)
    pltpu.sync_copy(dst_hbm.at[wid], dst_v)
    plsc.subcore_barrier()

    U = 5
    for i in range(U):
        pltpu.async_copy(y_hbm.at[src_v.at[i]], bufs[i], gsems[i])

    def body(t, carry):
        for i in range(U):
            j = U * t + i
            pltpu.make_async_copy(y_hbm.at[src_v.at[j]], bufs[i], gsems[i]).wait()
            pltpu.async_copy(bufs[i], acc.at[dst_v.at[j]], ssems[i], add=True)
        for i in range(U):
            jn = U * t + U + i
            pltpu.make_async_copy(bufs[i], acc.at[dst_v.at[jn]], ssems[i]).wait()
            pltpu.async_copy(y_hbm.at[src_v.at[jn]], bufs[i], gsems[i])
        return carry

    lax.fori_loop(0, C // U - 1, body, 0)
    for i in range(U):
        j = C - U + i
        pltpu.make_async_copy(y_hbm.at[src_v.at[j]], bufs[i], gsems[i]).wait()
        pltpu.async_copy(bufs[i], acc.at[dst_v.at[j]], ssems[i], add=True)
    for i in range(U):
        pltpu.make_async_copy(bufs[i], acc.at[dst_v.at[C - U + i]], ssems[i]).wait()
    plsc.subcore_barrier()
    pltpu.sync_copy(acc.at[pl.ds(base, RPT)], out_hbm.at[c, pl.ds(base, RPT)])


def _tc1_body(parts_ref, x_ref, w1_ref, dinv_ref, y1_ref):
    deg = parts_ref[0, :N, 0:1] + parts_ref[1, :N, 0:1] + 1.0
    dinv = lax.rsqrt(deg)
    dinv_ref[...] = dinv
    xw = jnp.dot(x_ref[...], w1_ref[...], preferred_element_type=jnp.float32)
    y1_ref[...] = dinv * xw


def _tc2_body(zp_ref, y1_ref, dinv_ref, b1_ref, y2_ref):
    dinv = dinv_ref[...]
    agg = zp_ref[0, :N, :] + zp_ref[1, :N, :] + y1_ref[...]
    h = jnp.maximum(dinv * agg + b1_ref[...], 0.0)
    y2_ref[...] = dinv * h


def _tc3_body(zp_ref, y2_ref, dinv_ref, w2_ref, b2_ref, out_ref):
    ah = dinv_ref[...] * (zp_ref[0, :N, :] + zp_ref[1, :N, :] + y2_ref[...])
    out_ref[...] = (
        jnp.dot(ah, w2_ref[...], preferred_element_type=jnp.float32)
        + b2_ref[...]
    )


_tc1 = pl.pallas_call(
    _tc1_body,
    out_shape=(
        jax.ShapeDtypeStruct((N, 1), jnp.float32),
        jax.ShapeDtypeStruct((N, D_HID), jnp.float32),
    ),
)
_tc2 = pl.pallas_call(
    _tc2_body,
    out_shape=jax.ShapeDtypeStruct((N, D_HID), jnp.float32),
)
_tc3 = pl.pallas_call(
    _tc3_body,
    out_shape=jax.ShapeDtypeStruct((N, D_OUT), jnp.float32),
)


def kernel(x, edge_index, W1, b1, W2, b2):
    src = edge_index[0].reshape(NW, C, K)
    dst = edge_index[1].reshape(NW, C, K)
    ones_col = jnp.ones((K, DW), jnp.float32)
    zeros_col = jnp.zeros((NP, DW), jnp.float32)
    zeros_rows = jnp.zeros((NP, D_HID), jnp.float32)

    deg_parts = _deg_sc(dst, ones_col, zeros_col)
    dinv, y1 = _tc1(deg_parts, x, W1)
    z1_parts = _scatter_sc(src, dst, y1, zeros_rows)
    y2 = _tc2(z1_parts, y1, dinv, b1.reshape(1, D_HID))
    z2_parts = _scatter_sc(src, dst, y2, zeros_rows)
    return _tc3(z2_parts, y2, dinv, W2, b2.reshape(1, D_OUT))


# speedup vs baseline: 42.4117x; 1.0286x over previous
import functools

import jax
import jax.numpy as jnp
from jax import lax
from jax.experimental import pallas as pl
from jax.experimental.pallas import tpu as pltpu
from jax.experimental.pallas import tpu_sc as plsc

N = 10000
E = 320000
D_IN = 128
D_HID = 64
D_OUT = 128

NC = 2
NS = 16
NW = NC * NS
EPW = E // NW
K = 80
C = EPW // K
NP = 10240
RPT = NP // NS

_mesh = plsc.VectorSubcoreMesh(core_axis_name="c", subcore_axis_name="s")
_sc_params = pltpu.CompilerParams(use_tc_tiling_on_sc=False)


DW = 16


@functools.partial(
    pl.kernel,
    out_type=jax.ShapeDtypeStruct((NC, NP, DW), jnp.float32),
    mesh=_mesh,
    scratch_types=[
        pltpu.VMEM((C, K), jnp.int32),
        pltpu.VMEM((K, DW), jnp.float32),
        pltpu.VMEM_SHARED((NP, DW), jnp.float32),
        pltpu.SemaphoreType.DMA,
    ],
    compiler_params=_sc_params,
)
def _deg_sc(dst_hbm, ones_hbm, zeros_hbm, out_hbm, dst_v, ones_v, acc, sem):
    c = lax.axis_index("c")
    s = lax.axis_index("s")
    wid = c * NS + s
    base = s * RPT
    pltpu.sync_copy(zeros_hbm.at[pl.ds(base, RPT)], acc.at[pl.ds(base, RPT)])
    pltpu.sync_copy(ones_hbm, ones_v)
    pltpu.sync_copy(dst_hbm.at[wid], dst_v)
    plsc.subcore_barrier()

    def body(j, carry):
        pltpu.async_copy(ones_v, acc.at[dst_v.at[j]], sem, add=True)
        return carry

    lax.fori_loop(0, C, body, 0)

    def drain(j, carry):
        pltpu.make_async_copy(ones_v, acc.at[dst_v.at[j]], sem).wait()
        return carry

    lax.fori_loop(0, C, drain, 0)
    plsc.subcore_barrier()
    pltpu.sync_copy(acc.at[pl.ds(base, RPT)], out_hbm.at[c, pl.ds(base, RPT)])


@functools.partial(
    pl.kernel,
    out_type=jax.ShapeDtypeStruct((NC, NP, D_HID), jnp.float32),
    mesh=_mesh,
    scratch_types=[
        pltpu.VMEM((C, K), jnp.int32),
        pltpu.VMEM((C, K), jnp.int32),
        [pltpu.VMEM((K, D_HID), jnp.float32) for _ in range(5)],
        pltpu.VMEM_SHARED((NP, D_HID), jnp.float32),
        [pltpu.SemaphoreType.DMA for _ in range(5)],
        [pltpu.SemaphoreType.DMA for _ in range(5)],
    ],
    compiler_params=_sc_params,
)
def _scatter_sc(src_hbm, dst_hbm, y_hbm, zeros_hbm, out_hbm,
                src_v, dst_v, bufs, acc, gsems, ssems):
    c = lax.axis_index("c")
    s = lax.axis_index("s")
    wid = c * NS + s
    base = s * RPT
    pltpu.sync_copy(zeros_hbm.at[pl.ds(base, RPT)], acc.at[pl.ds(base, RPT)])
    pltpu.sync_copy(src_hbm.at[wid], src_v)
    pltpu.sync_copy(dst_hbm.at[wid], dst_v)
    plsc.subcore_barrier()

    U = 5
    for i in range(U):
        pltpu.async_copy(y_hbm.at[src_v.at[i]], bufs[i], gsems[i])

    def body(t, carry):
        for i in range(U):
            j = U * t + i
            pltpu.make_async_copy(y_hbm.at[src_v.at[j]], bufs[i], gsems[i]).wait()
            pltpu.async_copy(bufs[i], acc.at[dst_v.at[j]], ssems[i], add=True)
        for i in range(U):
            jn = U * t + U + i
            pltpu.make_async_copy(bufs[i], acc.at[dst_v.at[jn]], ssems[i]).wait()
            pltpu.async_copy(y_hbm.at[src_v.at[jn]], bufs[i], gsems[i])
        return carry

    lax.fori_loop(0, C // U - 1, body, 0)
    for i in range(U):
        j = C - U + i
        pltpu.make_async_copy(y_hbm.at[src_v.at[j]], bufs[i], gsems[i]).wait()
        pltpu.async_copy(bufs[i], acc.at[dst_v.at[j]], ssems[i], add=True)
    for i in range(U):
        pltpu.make_async_copy(bufs[i], acc.at[dst_v.at[C - U + i]], ssems[i]).wait()
    plsc.subcore_barrier()
    pltpu.sync_copy(acc.at[pl.ds(base, RPT)], out_hbm.at[c, pl.ds(base, RPT)])


def _tc0_body(x_ref, w1_ref, xw_ref):
    xw_ref[...] = jnp.dot(
        x_ref[...], w1_ref[...], preferred_element_type=jnp.float32
    )


def _tc1_body(parts_ref, xw_ref, dinv_ref, y1_ref):
    deg = parts_ref[0, :N, 0:1] + parts_ref[1, :N, 0:1] + 1.0
    dinv = lax.rsqrt(deg)
    dinv_ref[...] = dinv
    y1_ref[...] = dinv * xw_ref[...]


def _tc2_body(zp_ref, y1_ref, dinv_ref, b1_ref, y2_ref):
    dinv = dinv_ref[...]
    agg = zp_ref[0, :N, :] + zp_ref[1, :N, :] + y1_ref[...]
    h = jnp.maximum(dinv * agg + b1_ref[...], 0.0)
    y2_ref[...] = dinv * h


def _tc3_body(zp_ref, y2_ref, dinv_ref, w2_ref, b2_ref, out_ref):
    ah = dinv_ref[...] * (zp_ref[0, :N, :] + zp_ref[1, :N, :] + y2_ref[...])
    out_ref[...] = (
        jnp.dot(ah, w2_ref[...], preferred_element_type=jnp.float32)
        + b2_ref[...]
    )


_tc0 = pl.pallas_call(
    _tc0_body,
    out_shape=jax.ShapeDtypeStruct((N, D_HID), jnp.float32),
)
_tc1 = pl.pallas_call(
    _tc1_body,
    out_shape=(
        jax.ShapeDtypeStruct((N, 1), jnp.float32),
        jax.ShapeDtypeStruct((N, D_HID), jnp.float32),
    ),
)
_tc2 = pl.pallas_call(
    _tc2_body,
    out_shape=jax.ShapeDtypeStruct((N, D_HID), jnp.float32),
)
_tc3 = pl.pallas_call(
    _tc3_body,
    out_shape=jax.ShapeDtypeStruct((N, D_OUT), jnp.float32),
)


def kernel(x, edge_index, W1, b1, W2, b2):
    src = edge_index[0].reshape(NW, C, K)
    dst = edge_index[1].reshape(NW, C, K)
    ones_col = jnp.ones((K, DW), jnp.float32)
    zeros_col = jnp.zeros((NP, DW), jnp.float32)
    zeros_rows = jnp.zeros((NP, D_HID), jnp.float32)

    xw = _tc0(x, W1)
    deg_parts = _deg_sc(dst, ones_col, zeros_col)
    dinv, y1 = _tc1(deg_parts, xw)
    z1_parts = _scatter_sc(src, dst, y1, zeros_rows)
    y2 = _tc2(z1_parts, y1, dinv, b1.reshape(1, D_HID))
    z2_parts = _scatter_sc(src, dst, y2, zeros_rows)
    return _tc3(z2_parts, y2, dinv, W2, b2.reshape(1, D_OUT))


# speedup vs baseline: 44.8216x; 1.0568x over previous
import functools

import jax
import jax.numpy as jnp
from jax import lax
from jax.experimental import pallas as pl
from jax.experimental.pallas import tpu as pltpu
from jax.experimental.pallas import tpu_sc as plsc

N = 10000
E = 320000
D_IN = 128
D_HID = 64
D_OUT = 128

NC = 2
NS = 16
NW = NC * NS
EPW = E // NW
K = 80
C = EPW // K
NP = 10240
RPT = NP // NS

_mesh = plsc.VectorSubcoreMesh(core_axis_name="c", subcore_axis_name="s")
_sc_params = pltpu.CompilerParams(use_tc_tiling_on_sc=False)


@functools.partial(
    pl.kernel,
    out_type=jax.ShapeDtypeStruct((NW, NP), jnp.float32),
    mesh=_mesh,
    scratch_types=[
        pltpu.VMEM((C, K), jnp.int32),
        pltpu.VMEM((NP,), jnp.float32),
    ],
    compiler_params=pltpu.CompilerParams(
        use_tc_tiling_on_sc=False, needs_layout_passes=False
    ),
)
def _deg_sc(dst_hbm, out_hbm, dst_v, hist):
    c = lax.axis_index("c")
    s = lax.axis_index("s")
    wid = c * NS + s
    zero16 = jnp.zeros((16,), jnp.float32)
    ones16 = jnp.ones((16,), jnp.float32)

    def zb(i, carry):
        hist[pl.ds(i * 16, 16)] = zero16
        return carry

    lax.fori_loop(0, NP // 16, zb, 0)
    pltpu.sync_copy(dst_hbm.at[wid], dst_v)

    def body(r, carry):
        for q in range(K // 16):
            ix = dst_v[r, pl.ds(q * 16, 16)]
            plsc.addupdate_scatter(hist, [ix], ones16)
        return carry

    lax.fori_loop(0, C, body, 0)
    pltpu.sync_copy(hist, out_hbm.at[wid])


@functools.partial(
    pl.kernel,
    out_type=jax.ShapeDtypeStruct((NC, NP, D_HID), jnp.float32),
    mesh=_mesh,
    scratch_types=[
        pltpu.VMEM((C, K), jnp.int32),
        pltpu.VMEM((C, K), jnp.int32),
        [pltpu.VMEM((K, D_HID), jnp.float32) for _ in range(5)],
        pltpu.VMEM_SHARED((NP, D_HID), jnp.float32),
        [pltpu.SemaphoreType.DMA for _ in range(5)],
        [pltpu.SemaphoreType.DMA for _ in range(5)],
    ],
    compiler_params=_sc_params,
)
def _scatter_sc(src_hbm, dst_hbm, y_hbm, zeros_hbm, out_hbm,
                src_v, dst_v, bufs, acc, gsems, ssems):
    c = lax.axis_index("c")
    s = lax.axis_index("s")
    wid = c * NS + s
    base = s * RPT
    pltpu.sync_copy(zeros_hbm.at[pl.ds(base, RPT)], acc.at[pl.ds(base, RPT)])
    pltpu.sync_copy(src_hbm.at[wid], src_v)
    pltpu.sync_copy(dst_hbm.at[wid], dst_v)
    plsc.subcore_barrier()

    U = 5
    for i in range(U):
        pltpu.async_copy(y_hbm.at[src_v.at[i]], bufs[i], gsems[i])

    def body(t, carry):
        for i in range(U):
            j = U * t + i
            pltpu.make_async_copy(y_hbm.at[src_v.at[j]], bufs[i], gsems[i]).wait()
            pltpu.async_copy(bufs[i], acc.at[dst_v.at[j]], ssems[i], add=True)
        for i in range(U):
            jn = U * t + U + i
            pltpu.make_async_copy(bufs[i], acc.at[dst_v.at[jn]], ssems[i]).wait()
            pltpu.async_copy(y_hbm.at[src_v.at[jn]], bufs[i], gsems[i])
        return carry

    lax.fori_loop(0, C // U - 1, body, 0)
    for i in range(U):
        j = C - U + i
        pltpu.make_async_copy(y_hbm.at[src_v.at[j]], bufs[i], gsems[i]).wait()
        pltpu.async_copy(bufs[i], acc.at[dst_v.at[j]], ssems[i], add=True)
    for i in range(U):
        pltpu.make_async_copy(bufs[i], acc.at[dst_v.at[C - U + i]], ssems[i]).wait()
    plsc.subcore_barrier()
    pltpu.sync_copy(acc.at[pl.ds(base, RPT)], out_hbm.at[c, pl.ds(base, RPT)])


def _tc0_body(x_ref, w1_ref, xw_ref):
    xw_ref[...] = jnp.dot(
        x_ref[...], w1_ref[...], preferred_element_type=jnp.float32
    )


def _tc1_body(parts_ref, xw_ref, dinv_ref, y1_ref):
    deg_col = lax.dot_general(
        parts_ref[...],
        jnp.ones((NW, 1), jnp.float32),
        (((0,), (0,)), ((), ())),
        preferred_element_type=jnp.float32,
    )
    deg = deg_col[:N] + 1.0
    dinv = lax.rsqrt(deg)
    dinv_ref[...] = dinv
    y1_ref[...] = dinv * xw_ref[...]


def _tc2_body(zp_ref, y1_ref, dinv_ref, b1_ref, y2_ref):
    dinv = dinv_ref[...]
    agg = zp_ref[0, :N, :] + zp_ref[1, :N, :] + y1_ref[...]
    h = jnp.maximum(dinv * agg + b1_ref[...], 0.0)
    y2_ref[...] = dinv * h


def _tc3_body(zp_ref, y2_ref, dinv_ref, w2_ref, b2_ref, out_ref):
    ah = dinv_ref[...] * (zp_ref[0, :N, :] + zp_ref[1, :N, :] + y2_ref[...])
    out_ref[...] = (
        jnp.dot(ah, w2_ref[...], preferred_element_type=jnp.float32)
        + b2_ref[...]
    )


_tc0 = pl.pallas_call(
    _tc0_body,
    out_shape=jax.ShapeDtypeStruct((N, D_HID), jnp.float32),
)
_tc1 = pl.pallas_call(
    _tc1_body,
    out_shape=(
        jax.ShapeDtypeStruct((N, 1), jnp.float32),
        jax.ShapeDtypeStruct((N, D_HID), jnp.float32),
    ),
)
_tc2 = pl.pallas_call(
    _tc2_body,
    out_shape=jax.ShapeDtypeStruct((N, D_HID), jnp.float32),
)
_tc3 = pl.pallas_call(
    _tc3_body,
    out_shape=jax.ShapeDtypeStruct((N, D_OUT), jnp.float32),
)


def kernel(x, edge_index, W1, b1, W2, b2):
    src = edge_index[0].reshape(NW, C, K)
    dst = edge_index[1].reshape(NW, C, K)
    zeros_rows = jnp.zeros((NP, D_HID), jnp.float32)

    xw = _tc0(x, W1)
    deg_parts = _deg_sc(dst)
    dinv, y1 = _tc1(deg_parts, xw)
    z1_parts = _scatter_sc(src, dst, y1, zeros_rows)
    y2 = _tc2(z1_parts, y1, dinv, b1.reshape(1, D_HID))
    z2_parts = _scatter_sc(src, dst, y2, zeros_rows)
    return _tc3(z2_parts, y2, dinv, W2, b2.reshape(1, D_OUT))
